# Initial kernel scaffold; baseline (speedup 1.0000x reference)
import jax
import jax.numpy as jnp
from jax.experimental import pallas as pl


def kernel(x, edge_index, edge_weight, W1, b1, ln1_w, ln1_b, W2, b2, ln2_w, ln2_b, rW1, rb1, ln3_w, ln3_b, rW2, rb2):
    raise NotImplementedError("write your pallas kernel here")


# speedup vs baseline: 10.1886x; 10.1886x over previous
import functools

import jax
import jax.numpy as jnp
from jax import lax
from jax.experimental import pallas as pl
from jax.experimental.pallas import tpu as pltpu
from jax.experimental.pallas import tpu_sc as plsc

N = 10000
NP = 10240
D = 128
NTILES = 32
G = 128
ROWS_PER_TILE = NP // 16

_mesh = plsc.VectorSubcoreMesh(core_axis_name="c", subcore_axis_name="s")


def _num_chunks(ep):
    return ep // (NTILES * G)


def _make_deg_kernel(ep):
    nch = _num_chunks(ep)

    @functools.partial(
        pl.kernel,
        out_type=jax.ShapeDtypeStruct((2, NP, D), jnp.float32),
        scratch_types=[
            pltpu.VMEM_SHARED((NP, D), jnp.float32),
            pltpu.VMEM((1, G), jnp.int32),
            pltpu.VMEM((G,), jnp.float32),
            pltpu.VMEM((G, D), jnp.float32),
        ],
        mesh=_mesh,
    )
    def deg_kernel(dst_hbm, ew_hbm, zeros_hbm, out_hbm, dacc, dstbuf, ewbuf, rows):
        c = lax.axis_index("c")
        s = lax.axis_index("s")
        tile = c * 16 + s
        base_e = tile * nch * G
        rbase = s * ROWS_PER_TILE

        pltpu.sync_copy(zeros_hbm.at[pl.ds(rbase, ROWS_PER_TILE)],
                        dacc.at[pl.ds(rbase, ROWS_PER_TILE)])
        plsc.subcore_barrier()

        def chunk_body(k, _):
            off = base_e + k * G
            pltpu.sync_copy(dst_hbm.at[pl.ds(off, G)], dstbuf.at[0])
            pltpu.sync_copy(ew_hbm.at[pl.ds(off, G)], ewbuf)

            def fill_body(i, _):
                ewv = ewbuf[pl.ds(i * 16, 16)]
                for l in range(16):
                    v = jnp.full((16,), ewv[l], jnp.float32)
                    for j in range(D // 16):
                        rows[i * 16 + l, pl.ds(j * 16, 16)] = v
                return 0

            lax.fori_loop(0, G // 16, fill_body, 0)
            pltpu.sync_copy(rows, dacc.at[dstbuf.at[0]], add=True)
            return 0

        lax.fori_loop(0, nch, chunk_body, 0)
        plsc.subcore_barrier()
        pltpu.sync_copy(dacc.at[pl.ds(rbase, ROWS_PER_TILE)],
                        out_hbm.at[c, pl.ds(rbase, ROWS_PER_TILE)])

    return deg_kernel


def _make_spmm_kernel(ep):
    nch = _num_chunks(ep)

    @functools.partial(
        pl.kernel,
        out_type=jax.ShapeDtypeStruct((2, NP, D), jnp.float32),
        scratch_types=[
            pltpu.VMEM_SHARED((NP, D), jnp.float32),
            pltpu.VMEM((1, G), jnp.int32),
            pltpu.VMEM((1, G), jnp.int32),
            pltpu.VMEM((G,), jnp.float32),
            pltpu.VMEM((1, G, D), jnp.float32),
            pltpu.SemaphoreType.DMA,
        ],
        mesh=_mesh,
    )
    def spmm_kernel(hp_hbm, src_hbm, dst_hbm, ew_hbm, zeros_hbm, out_hbm,
                    acc, srcbuf, dstbuf, ewbuf, rows, sem):
        c = lax.axis_index("c")
        s = lax.axis_index("s")
        tile = c * 16 + s
        base_e = tile * nch * G
        rbase = s * ROWS_PER_TILE

        pltpu.sync_copy(zeros_hbm.at[pl.ds(rbase, ROWS_PER_TILE)],
                        acc.at[pl.ds(rbase, ROWS_PER_TILE)])
        plsc.subcore_barrier()


        def chunk_body(k, _):
            off = base_e + k * G
            pltpu.sync_copy(src_hbm.at[pl.ds(off, G)], srcbuf.at[0])
            pltpu.sync_copy(dst_hbm.at[pl.ds(off, G)], dstbuf.at[0])
            pltpu.sync_copy(ew_hbm.at[pl.ds(off, G)], ewbuf)
            pltpu.async_copy(hp_hbm.at[srcbuf.at[0]], rows.at[0], sem).wait()

            def scale_body(i, _):
                ewv = ewbuf[pl.ds(i * 16, 16)]
                for l in range(16):
                    v = jnp.full((16,), ewv[l], jnp.float32)
                    g = i * 16 + l
                    for j in range(D // 16):
                        sl = pl.ds(j * 16, 16)
                        rows[0, g, sl] = rows[0, g, sl] * v
                return 0

            lax.fori_loop(0, G // 16, scale_body, 0)
            pltpu.sync_copy(rows.at[0], acc.at[dstbuf.at[0]], add=True)
            return 0

        lax.fori_loop(0, nch, chunk_body, 0)
        plsc.subcore_barrier()
        pltpu.sync_copy(acc.at[pl.ds(rbase, ROWS_PER_TILE)],
                        out_hbm.at[c, pl.ds(rbase, ROWS_PER_TILE)])

    return spmm_kernel


_BLK = 512
_GRID = NP // _BLK

_row_spec = pl.BlockSpec((_BLK, D), lambda i: (i, 0))
_col_spec = pl.BlockSpec((_BLK, 1), lambda i: (i, 0))
_full_spec = lambda shape: pl.BlockSpec(shape, lambda i: (0,) * len(shape))


def _ln(x, w, b, eps=1e-5):
    mu = jnp.mean(x, axis=-1, keepdims=True)
    var = jnp.mean((x - mu) ** 2, axis=-1, keepdims=True)
    return (x - mu) * lax.rsqrt(var + eps) * w + b


def _tc_pre_body(x_ref, w1_ref, d0_ref, d1_ref, t1_ref, hp1_ref, dinv_ref):
    deg = d0_ref[...] + d1_ref[...] + 1.0
    dv = lax.rsqrt(deg)
    t = jnp.dot(x_ref[...], w1_ref[...], preferred_element_type=jnp.float32)
    t1_ref[...] = t
    hp1_ref[...] = t * dv
    dinv_ref[...] = dv


_tc_pre = pl.pallas_call(
    _tc_pre_body,
    grid=(_GRID,),
    in_specs=[_row_spec, _full_spec((D, D)), _col_spec, _col_spec],
    out_specs=[_row_spec, _row_spec, _col_spec],
    out_shape=[
        jax.ShapeDtypeStruct((NP, D), jnp.float32),
        jax.ShapeDtypeStruct((NP, D), jnp.float32),
        jax.ShapeDtypeStruct((NP, 1), jnp.float32),
    ],
)


def _tc_mid_body(a0_ref, a1_ref, t1_ref, dinv_ref, b1_ref, lw_ref, lb_ref,
                 w2_ref, t2_ref, hp2_ref):
    dv = dinv_ref[...]
    pre = dv * (a0_ref[...] + a1_ref[...]) + dv * dv * t1_ref[...] + b1_ref[...]
    h1 = jnp.maximum(_ln(pre, lw_ref[...], lb_ref[...]), 0.0)
    t2 = jnp.dot(h1, w2_ref[...], preferred_element_type=jnp.float32)
    t2_ref[...] = t2
    hp2_ref[...] = t2 * dv


_tc_mid = pl.pallas_call(
    _tc_mid_body,
    grid=(_GRID,),
    in_specs=[_row_spec, _row_spec, _row_spec, _col_spec,
              _full_spec((1, D)), _full_spec((1, D)), _full_spec((1, D)),
              _full_spec((D, D))],
    out_specs=[_row_spec, _row_spec],
    out_shape=[
        jax.ShapeDtypeStruct((NP, D), jnp.float32),
        jax.ShapeDtypeStruct((NP, D), jnp.float32),
    ],
)


def _tc_post_body(a0_ref, a1_ref, t2_ref, dinv_ref, b2_ref, l2w_ref, l2b_ref,
                  rw1_ref, rb1_ref, l3w_ref, l3b_ref, rw2_ref, rb2_ref, out_ref):
    dv = dinv_ref[...]
    pre = dv * (a0_ref[...] + a1_ref[...]) + dv * dv * t2_ref[...] + b2_ref[...]
    h2 = jnp.maximum(_ln(pre, l2w_ref[...], l2b_ref[...]), 0.0)
    r = jnp.dot(h2, rw1_ref[...], preferred_element_type=jnp.float32) + rb1_ref[...]
    r = jnp.maximum(_ln(r, l3w_ref[...], l3b_ref[...]), 0.0)
    out_ref[...] = jnp.dot(r, rw2_ref[...], preferred_element_type=jnp.float32) + rb2_ref[...]


_tc_post = pl.pallas_call(
    _tc_post_body,
    grid=(_GRID,),
    in_specs=[_row_spec, _row_spec, _row_spec, _col_spec,
              _full_spec((1, D)), _full_spec((1, D)), _full_spec((1, D)),
              _full_spec((D, 32)), _full_spec((1, 32)), _full_spec((1, 32)),
              _full_spec((1, 32)), _full_spec((32, D)), _full_spec((1, D))],
    out_specs=_row_spec,
    out_shape=jax.ShapeDtypeStruct((NP, D), jnp.float32),
)


def kernel(x, edge_index, edge_weight, W1, b1, ln1_w, ln1_b, W2, b2, ln2_w,
           ln2_b, rW1, rb1, ln3_w, ln3_b, rW2, rb2):
    e = edge_index.shape[1]
    ep = -(-e // (NTILES * G)) * (NTILES * G)
    pad = ep - e

    src = edge_index[0].astype(jnp.int32)
    dst = edge_index[1].astype(jnp.int32)
    ew = edge_weight.astype(jnp.float32)
    if pad:
        padidx = (jnp.arange(pad, dtype=jnp.int32) * 37) % N
        src = jnp.concatenate([src, padidx])
        dst = jnp.concatenate([dst, padidx])
        ew = jnp.concatenate([ew, jnp.zeros((pad,), jnp.float32)])

    xp = jnp.pad(x.astype(jnp.float32), ((0, NP - N), (0, 0)))
    zrows = jnp.zeros((NP, D), jnp.float32)

    deg_k = _make_deg_kernel(ep)
    spmm_k = _make_spmm_kernel(ep)

    degp = deg_k(dst, ew, zrows)
    d0 = degp[0, :, 0:1]
    d1 = degp[1, :, 0:1]

    t1, hp1, dinv = _tc_pre(xp, W1, d0, d1)

    agg1 = spmm_k(hp1, src, dst, ew, zrows)
    t2, hp2 = _tc_mid(agg1[0], agg1[1], t1, dinv, b1.reshape(1, D),
                      ln1_w.reshape(1, D), ln1_b.reshape(1, D), W2)

    agg2 = spmm_k(hp2, src, dst, ew, zrows)
    out = _tc_post(agg2[0], agg2[1], t2, dinv, b2.reshape(1, D),
                   ln2_w.reshape(1, D), ln2_b.reshape(1, D),
                   rW1, rb1.reshape(1, 32), ln3_w.reshape(1, 32),
                   ln3_b.reshape(1, 32), rW2, rb2.reshape(1, D))
    return out[:N]


# speedup vs baseline: 15.6219x; 1.5333x over previous
import functools

import jax
import jax.numpy as jnp
from jax import lax
from jax.experimental import pallas as pl
from jax.experimental.pallas import tpu as pltpu
from jax.experimental.pallas import tpu_sc as plsc

N = 10000
NP = 10240
D = 128
NTILES = 32
G = 128
ROWS_PER_TILE = NP // 16

_mesh = plsc.VectorSubcoreMesh(core_axis_name="c", subcore_axis_name="s")


def _num_chunks(ep):
    nch = ep // (NTILES * G)
    assert nch % 4 == 0
    return nch


def _make_deg_kernel(ep):
    nch = _num_chunks(ep)

    @functools.partial(
        pl.kernel,
        out_type=jax.ShapeDtypeStruct((2, NP, D), jnp.float32),
        scratch_types=[
            pltpu.VMEM_SHARED((NP, D), jnp.float32),
            pltpu.VMEM((2, G), jnp.int32),
            pltpu.VMEM((2, G), jnp.float32),
            pltpu.VMEM((2, G), jnp.int32),
            pltpu.VMEM((2, G, D), jnp.float32),
            pltpu.SemaphoreType.DMA,
            pltpu.SemaphoreType.DMA,
            pltpu.SemaphoreType.DMA,
            pltpu.SemaphoreType.DMA,
        ],
        mesh=_mesh,
    )
    def deg_kernel(dst_hbm, ew_hbm, zeros_hbm, out_hbm, dacc, dstb, ewb,
                   sidx, rows, i0, i1, s0, s1):
        isem = (i0, i1)
        ssem = (s0, s1)
        c = lax.axis_index("c")
        s = lax.axis_index("s")
        tile = c * 16 + s
        rbase = s * ROWS_PER_TILE

        pltpu.sync_copy(zeros_hbm.at[pl.ds(rbase, ROWS_PER_TILE)],
                        dacc.at[pl.ds(rbase, ROWS_PER_TILE)])
        pltpu.sync_copy(zeros_hbm.at[pl.ds(0, G)], rows.at[0])
        pltpu.sync_copy(zeros_hbm.at[pl.ds(0, G)], rows.at[1])
        base_e = tile * nch * G
        pltpu.async_copy(dst_hbm.at[pl.ds(tile * nch, 1)], dstb.at[pl.ds(0, 1)],
                         i0)
        pltpu.async_copy(ew_hbm.at[pl.ds(base_e, G)], ewb.at[0], i0)
        plsc.subcore_barrier()

        def fill(b, _unused):
            def grp(i, _):
                ewv = ewb[b, pl.ds(i * 16, 16)]
                for l in range(16):
                    rows[b, i * 16 + l, pl.ds(0, 16)] = jnp.full(
                        (16,), ewv[l], jnp.float32)
                return 0

            lax.fori_loop(0, G // 16, grp, 0)

        def pair_body(p, _):
            for sub in (0, 1):
                k = p * 2 + sub
                bn = 1 - sub
                pltpu.make_async_copy(dst_hbm.at[pl.ds(tile * nch + k, 1)],
                                      dstb.at[pl.ds(sub, 1)], isem[sub]).wait()
                pltpu.make_async_copy(ew_hbm.at[pl.ds(base_e + k * G, G)],
                                      ewb.at[sub], isem[sub]).wait()

                @pl.when(k >= 2)
                def _():
                    pltpu.make_async_copy(
                        rows.at[sub], dacc.at[sidx.at[sub]], ssem[sub]).wait()

                fill(sub, k)
                for j in range(G // 16):
                    sidx[sub, pl.ds(j * 16, 16)] = dstb[sub, pl.ds(j * 16, 16)]
                pltpu.async_copy(rows.at[sub], dacc.at[sidx.at[sub]],
                                 ssem[sub], add=True)

                @pl.when(k + 1 < nch)
                def _():
                    pltpu.async_copy(
                        dst_hbm.at[pl.ds(tile * nch + k + 1, 1)],
                        dstb.at[pl.ds(bn, 1)], isem[bn])
                    pltpu.async_copy(ew_hbm.at[pl.ds(base_e + (k + 1) * G, G)],
                                     ewb.at[bn], isem[bn])
            return 0

        lax.fori_loop(0, nch // 2, pair_body, 0)
        pltpu.make_async_copy(rows.at[0], dacc.at[sidx.at[0]], ssem[0]).wait()
        pltpu.make_async_copy(rows.at[1], dacc.at[sidx.at[1]], ssem[1]).wait()
        plsc.subcore_barrier()
        pltpu.sync_copy(dacc.at[pl.ds(rbase, ROWS_PER_TILE)],
                        out_hbm.at[c, pl.ds(rbase, ROWS_PER_TILE)])

    return deg_kernel


def _make_spmm_kernel(ep):
    nch = _num_chunks(ep)

    @functools.partial(
        pl.kernel,
        out_type=jax.ShapeDtypeStruct((2, NP, D), jnp.float32),
        scratch_types=[
            pltpu.VMEM_SHARED((NP, D), jnp.float32),
            pltpu.VMEM((2, G), jnp.int32),
            pltpu.VMEM((2, G), jnp.int32),
            pltpu.VMEM((2, G), jnp.float32),
            pltpu.VMEM((2, G, D), jnp.float32),
            pltpu.SemaphoreType.DMA,
            pltpu.SemaphoreType.DMA,
            pltpu.SemaphoreType.DMA,
            pltpu.SemaphoreType.DMA,
        ],
        mesh=_mesh,
    )
    def spmm_kernel(hp_hbm, src_hbm, dst_hbm, ew_hbm, zeros_hbm, out_hbm,
                    acc, srcb, dstb, ewb, rows, i0, i1, g0, g1):
        isem = (i0, i1)
        gsem = (g0, g1)
        c = lax.axis_index("c")
        s = lax.axis_index("s")
        tile = c * 16 + s
        base_c = tile * nch
        base_e = tile * nch * G
        rbase = s * ROWS_PER_TILE

        def fetch_idx(k, b, sem):
            pltpu.async_copy(src_hbm.at[pl.ds(base_c + k, 1)],
                             srcb.at[pl.ds(b, 1)], sem)
            pltpu.async_copy(dst_hbm.at[pl.ds(base_c + k, 1)],
                             dstb.at[pl.ds(b, 1)], sem)
            pltpu.async_copy(ew_hbm.at[pl.ds(base_e + k * G, G)],
                             ewb.at[b], sem)

        def wait_idx(k, b, sem):
            pltpu.make_async_copy(src_hbm.at[pl.ds(base_c + k, 1)],
                                  srcb.at[pl.ds(b, 1)], sem).wait()
            pltpu.make_async_copy(dst_hbm.at[pl.ds(base_c + k, 1)],
                                  dstb.at[pl.ds(b, 1)], sem).wait()
            pltpu.make_async_copy(ew_hbm.at[pl.ds(base_e + k * G, G)],
                                  ewb.at[b], sem).wait()

        pltpu.sync_copy(zeros_hbm.at[pl.ds(rbase, ROWS_PER_TILE)],
                        acc.at[pl.ds(rbase, ROWS_PER_TILE)])
        plsc.subcore_barrier()

        fetch_idx(0, 0, i0)
        wait_idx(0, 0, i0)
        pltpu.async_copy(hp_hbm.at[srcb.at[0]], rows.at[0], g0)
        fetch_idx(1, 1, i1)

        def scale(b, _unused):
            def grp(i, _):
                ewv = ewb[b, pl.ds(i * 16, 16)]
                for l in range(16):
                    v = jnp.full((16,), ewv[l], jnp.float32)
                    g = i * 16 + l
                    for j in range(D // 16):
                        sl = pl.ds(j * 16, 16)
                        rows[b, g, sl] = rows[b, g, sl] * v
                return 0

            lax.fori_loop(0, G // 16, grp, 0)

        def pair_body(p, _):
            for sub in (0, 1):
                k = p * 2 + sub
                bn = 1 - sub
                pltpu.make_async_copy(hp_hbm.at[srcb.at[sub]], rows.at[sub],
                                      gsem[sub]).wait()
                scale(sub, k)
                pltpu.sync_copy(rows.at[sub], acc.at[dstb.at[sub]], add=True)

                @pl.when(k + 1 < nch)
                def _():
                    wait_idx(k + 1, bn, isem[bn])
                    pltpu.async_copy(hp_hbm.at[srcb.at[bn]], rows.at[bn],
                                     gsem[bn])

                @pl.when(k + 2 < nch)
                def _():
                    fetch_idx(k + 2, sub, isem[sub])
            return 0

        lax.fori_loop(0, nch // 2, pair_body, 0)
        plsc.subcore_barrier()
        pltpu.sync_copy(acc.at[pl.ds(rbase, ROWS_PER_TILE)],
                        out_hbm.at[c, pl.ds(rbase, ROWS_PER_TILE)])

    return spmm_kernel


_BLK = 512
_GRID = NP // _BLK

_row_spec = pl.BlockSpec((_BLK, D), lambda i: (i, 0))
_col_spec = pl.BlockSpec((_BLK, 1), lambda i: (i, 0))
_pair_spec = pl.BlockSpec((2, _BLK, D), lambda i: (0, i, 0))
_full_spec = lambda shape: pl.BlockSpec(shape, lambda i: (0,) * len(shape))


def _ln(x, w, b, eps=1e-5):
    mu = jnp.mean(x, axis=-1, keepdims=True)
    var = jnp.mean((x - mu) ** 2, axis=-1, keepdims=True)
    return (x - mu) * lax.rsqrt(var + eps) * w + b


def _tc_pre_body(x_ref, w1_ref, dp_ref, t1_ref, hp1_ref, dinv_ref):
    deg = dp_ref[0, :, 0:1] + dp_ref[1, :, 0:1] + 1.0
    dv = lax.rsqrt(deg)
    t = jnp.dot(x_ref[...], w1_ref[...], preferred_element_type=jnp.float32)
    t1_ref[...] = t
    hp1_ref[...] = t * dv
    dinv_ref[...] = dv


_tc_pre = pl.pallas_call(
    _tc_pre_body,
    grid=(_GRID,),
    in_specs=[_row_spec, _full_spec((D, D)), _pair_spec],
    out_specs=[_row_spec, _row_spec, _col_spec],
    out_shape=[
        jax.ShapeDtypeStruct((NP, D), jnp.float32),
        jax.ShapeDtypeStruct((NP, D), jnp.float32),
        jax.ShapeDtypeStruct((NP, 1), jnp.float32),
    ],
)


def _tc_mid_body(ap_ref, t1_ref, dinv_ref, b1_ref, lw_ref, lb_ref,
                 w2_ref, t2_ref, hp2_ref):
    dv = dinv_ref[...]
    pre = dv * (ap_ref[0] + ap_ref[1]) + dv * dv * t1_ref[...] + b1_ref[...]
    h1 = jnp.maximum(_ln(pre, lw_ref[...], lb_ref[...]), 0.0)
    t2 = jnp.dot(h1, w2_ref[...], preferred_element_type=jnp.float32)
    t2_ref[...] = t2
    hp2_ref[...] = t2 * dv


_tc_mid = pl.pallas_call(
    _tc_mid_body,
    grid=(_GRID,),
    in_specs=[_pair_spec, _row_spec, _col_spec,
              _full_spec((1, D)), _full_spec((1, D)), _full_spec((1, D)),
              _full_spec((D, D))],
    out_specs=[_row_spec, _row_spec],
    out_shape=[
        jax.ShapeDtypeStruct((NP, D), jnp.float32),
        jax.ShapeDtypeStruct((NP, D), jnp.float32),
    ],
)


def _tc_post_body(ap_ref, t2_ref, dinv_ref, b2_ref, l2w_ref, l2b_ref,
                  rw1_ref, rb1_ref, l3w_ref, l3b_ref, rw2_ref, rb2_ref, out_ref):
    dv = dinv_ref[...]
    pre = dv * (ap_ref[0] + ap_ref[1]) + dv * dv * t2_ref[...] + b2_ref[...]
    h2 = jnp.maximum(_ln(pre, l2w_ref[...], l2b_ref[...]), 0.0)
    r = jnp.dot(h2, rw1_ref[...], preferred_element_type=jnp.float32) + rb1_ref[...]
    r = jnp.maximum(_ln(r, l3w_ref[...], l3b_ref[...]), 0.0)
    out_ref[...] = jnp.dot(r, rw2_ref[...], preferred_element_type=jnp.float32) + rb2_ref[...]


_tc_post = pl.pallas_call(
    _tc_post_body,
    grid=(_GRID,),
    in_specs=[_pair_spec, _row_spec, _col_spec,
              _full_spec((1, D)), _full_spec((1, D)), _full_spec((1, D)),
              _full_spec((D, 32)), _full_spec((1, 32)), _full_spec((1, 32)),
              _full_spec((1, 32)), _full_spec((32, D)), _full_spec((1, D))],
    out_specs=_row_spec,
    out_shape=jax.ShapeDtypeStruct((NP, D), jnp.float32),
)


def kernel(x, edge_index, edge_weight, W1, b1, ln1_w, ln1_b, W2, b2, ln2_w,
           ln2_b, rW1, rb1, ln3_w, ln3_b, rW2, rb2):
    e = edge_index.shape[1]
    step = NTILES * G * 4
    ep = -(-e // step) * step
    pad = ep - e

    src = edge_index[0].astype(jnp.int32)
    dst = edge_index[1].astype(jnp.int32)
    ew = edge_weight.astype(jnp.float32)
    if pad:
        padidx = (jnp.arange(pad, dtype=jnp.int32) * 37) % N
        src = jnp.concatenate([src, padidx])
        dst = jnp.concatenate([dst, padidx])
        ew = jnp.concatenate([ew, jnp.zeros((pad,), jnp.float32)])
    src2d = src.reshape(ep // G, G)
    dst2d = dst.reshape(ep // G, G)

    xp = jnp.pad(x.astype(jnp.float32), ((0, NP - N), (0, 0)))
    zrows = jnp.zeros((NP, D), jnp.float32)

    deg_k = _make_deg_kernel(ep)
    spmm_k = _make_spmm_kernel(ep)

    degp = deg_k(dst2d, ew, zrows)
    t1, hp1, dinv = _tc_pre(xp, W1, degp)

    agg1 = spmm_k(hp1, src2d, dst2d, ew, zrows)
    t2, hp2 = _tc_mid(agg1, t1, dinv, b1.reshape(1, D),
                      ln1_w.reshape(1, D), ln1_b.reshape(1, D), W2)

    agg2 = spmm_k(hp2, src2d, dst2d, ew, zrows)
    out = _tc_post(agg2, t2, dinv, b2.reshape(1, D),
                   ln2_w.reshape(1, D), ln2_b.reshape(1, D),
                   rW1, rb1.reshape(1, 32), ln3_w.reshape(1, 32),
                   ln3_b.reshape(1, 32), rW2, rb2.reshape(1, D))
    return out[:N]


# speedup vs baseline: 18.0945x; 1.1583x over previous
import functools

import jax
import jax.numpy as jnp
from jax import lax
from jax.experimental import pallas as pl
from jax.experimental.pallas import tpu as pltpu
from jax.experimental.pallas import tpu_sc as plsc

N = 10000
NP = 10240
D = 128
NTILES = 32
G = 112
ROWS_PER_TILE = NP // 16

_mesh = plsc.VectorSubcoreMesh(core_axis_name="c", subcore_axis_name="s")


def _num_chunks(ep):
    nch = ep // (NTILES * G)
    assert nch % 6 == 0
    return nch


def _make_deg_kernel(ep):
    nch = _num_chunks(ep)

    @functools.partial(
        pl.kernel,
        out_type=jax.ShapeDtypeStruct((2, NP, D), jnp.float32),
        scratch_types=[
            pltpu.VMEM_SHARED((NP, D), jnp.float32),
            pltpu.VMEM((2, G), jnp.int32),
            pltpu.VMEM((2, G), jnp.float32),
            pltpu.VMEM((2, G), jnp.int32),
            pltpu.VMEM((2, G, D), jnp.float32),
            pltpu.SemaphoreType.DMA,
            pltpu.SemaphoreType.DMA,
            pltpu.SemaphoreType.DMA,
            pltpu.SemaphoreType.DMA,
        ],
        mesh=_mesh,
    )
    def deg_kernel(dst_hbm, ew_hbm, zeros_hbm, out_hbm, dacc, dstb, ewb,
                   sidx, rows, i0, i1, s0, s1):
        isem = (i0, i1)
        ssem = (s0, s1)
        c = lax.axis_index("c")
        s = lax.axis_index("s")
        tile = c * 16 + s
        rbase = s * ROWS_PER_TILE

        pltpu.sync_copy(zeros_hbm.at[pl.ds(rbase, ROWS_PER_TILE)],
                        dacc.at[pl.ds(rbase, ROWS_PER_TILE)])
        pltpu.sync_copy(zeros_hbm.at[pl.ds(0, G)], rows.at[0])
        pltpu.sync_copy(zeros_hbm.at[pl.ds(0, G)], rows.at[1])
        base_e = tile * nch * G
        pltpu.async_copy(dst_hbm.at[pl.ds(tile * nch, 1)], dstb.at[pl.ds(0, 1)],
                         i0)
        pltpu.async_copy(ew_hbm.at[pl.ds(base_e, G)], ewb.at[0], i0)
        plsc.subcore_barrier()

        def fill(b, _unused):
            def grp(i, _):
                ewv = ewb[b, pl.ds(i * 16, 16)]
                for l in range(16):
                    rows[b, i * 16 + l, pl.ds(0, 16)] = jnp.full(
                        (16,), ewv[l], jnp.float32)
                return 0

            lax.fori_loop(0, G // 16, grp, 0)

        def pair_body(p, _):
            for sub in (0, 1):
                k = p * 2 + sub
                bn = 1 - sub
                pltpu.make_async_copy(dst_hbm.at[pl.ds(tile * nch + k, 1)],
                                      dstb.at[pl.ds(sub, 1)], isem[sub]).wait()
                pltpu.make_async_copy(ew_hbm.at[pl.ds(base_e + k * G, G)],
                                      ewb.at[sub], isem[sub]).wait()

                @pl.when(k >= 2)
                def _():
                    pltpu.make_async_copy(
                        rows.at[sub], dacc.at[sidx.at[sub]], ssem[sub]).wait()

                fill(sub, k)
                for j in range(G // 16):
                    sidx[sub, pl.ds(j * 16, 16)] = dstb[sub, pl.ds(j * 16, 16)]
                pltpu.async_copy(rows.at[sub], dacc.at[sidx.at[sub]],
                                 ssem[sub], add=True)

                @pl.when(k + 1 < nch)
                def _():
                    pltpu.async_copy(
                        dst_hbm.at[pl.ds(tile * nch + k + 1, 1)],
                        dstb.at[pl.ds(bn, 1)], isem[bn])
                    pltpu.async_copy(ew_hbm.at[pl.ds(base_e + (k + 1) * G, G)],
                                     ewb.at[bn], isem[bn])
            return 0

        lax.fori_loop(0, nch // 2, pair_body, 0)
        pltpu.make_async_copy(rows.at[0], dacc.at[sidx.at[0]], ssem[0]).wait()
        pltpu.make_async_copy(rows.at[1], dacc.at[sidx.at[1]], ssem[1]).wait()
        plsc.subcore_barrier()
        pltpu.sync_copy(dacc.at[pl.ds(rbase, ROWS_PER_TILE)],
                        out_hbm.at[c, pl.ds(rbase, ROWS_PER_TILE)])

    return deg_kernel


def _make_spmm_kernel(ep):
    nch = _num_chunks(ep)

    @functools.partial(
        pl.kernel,
        out_type=jax.ShapeDtypeStruct((2, NP, D), jnp.float32),
        scratch_types=[
            pltpu.VMEM_SHARED((NP, D), jnp.float32),
            pltpu.VMEM((3, G), jnp.int32),
            pltpu.VMEM((3, G), jnp.int32),
            pltpu.VMEM((3, G), jnp.float32),
            pltpu.VMEM((3, G), jnp.int32),
            pltpu.VMEM((3, G, D), jnp.float32),
            pltpu.SemaphoreType.DMA,
            pltpu.SemaphoreType.DMA,
            pltpu.SemaphoreType.DMA,
            pltpu.SemaphoreType.DMA,
            pltpu.SemaphoreType.DMA,
            pltpu.SemaphoreType.DMA,
            pltpu.SemaphoreType.DMA,
            pltpu.SemaphoreType.DMA,
            pltpu.SemaphoreType.DMA,
        ],
        mesh=_mesh,
    )
    def spmm_kernel(hp_hbm, src_hbm, dst_hbm, ew_hbm, zeros_hbm, out_hbm,
                    acc, srcb, dstb, ewb, sidx, rows,
                    i0, i1, i2, g0, g1, g2, s0, s1, s2):
        isem = (i0, i1, i2)
        gsem = (g0, g1, g2)
        ssem = (s0, s1, s2)
        c = lax.axis_index("c")
        s = lax.axis_index("s")
        tile = c * 16 + s
        base_c = tile * nch
        base_e = tile * nch * G
        rbase = s * ROWS_PER_TILE

        def fetch_idx(k, b, sem):
            pltpu.async_copy(src_hbm.at[pl.ds(base_c + k, 1)],
                             srcb.at[pl.ds(b, 1)], sem)
            pltpu.async_copy(dst_hbm.at[pl.ds(base_c + k, 1)],
                             dstb.at[pl.ds(b, 1)], sem)
            pltpu.async_copy(ew_hbm.at[pl.ds(base_e + k * G, G)],
                             ewb.at[b], sem)

        def wait_idx(k, b, sem):
            pltpu.make_async_copy(src_hbm.at[pl.ds(base_c + k, 1)],
                                  srcb.at[pl.ds(b, 1)], sem).wait()
            pltpu.make_async_copy(dst_hbm.at[pl.ds(base_c + k, 1)],
                                  dstb.at[pl.ds(b, 1)], sem).wait()
            pltpu.make_async_copy(ew_hbm.at[pl.ds(base_e + k * G, G)],
                                  ewb.at[b], sem).wait()

        pltpu.sync_copy(zeros_hbm.at[pl.ds(rbase, ROWS_PER_TILE)],
                        acc.at[pl.ds(rbase, ROWS_PER_TILE)])
        plsc.subcore_barrier()

        fetch_idx(0, 0, i0)
        wait_idx(0, 0, i0)
        pltpu.async_copy(hp_hbm.at[srcb.at[0]], rows.at[0], g0)
        fetch_idx(1, 1, i1)

        def scale(b, _unused):
            def grp(i, _):
                ewv = ewb[b, pl.ds(i * 16, 16)]
                for l in range(16):
                    v = jnp.full((16,), ewv[l], jnp.float32)
                    g = i * 16 + l
                    for j in range(D // 16):
                        sl = pl.ds(j * 16, 16)
                        rows[b, g, sl] = rows[b, g, sl] * v
                return 0

            lax.fori_loop(0, G // 16, grp, 0)

        def tri_body(p, _):
            for sub in range(3):
                k = p * 3 + sub
                bg = (sub + 1) % 3
                bf = (sub + 2) % 3
                pltpu.make_async_copy(hp_hbm.at[srcb.at[sub]], rows.at[sub],
                                      gsem[sub]).wait()
                scale(sub, k)
                for j in range(G // 16):
                    sidx[sub, pl.ds(j * 16, 16)] = dstb[sub, pl.ds(j * 16, 16)]
                pltpu.async_copy(rows.at[sub], acc.at[sidx.at[sub]],
                                 ssem[sub], add=True)

                @pl.when(k + 1 < nch)
                def _():
                    wait_idx(k + 1, bg, isem[bg])

                    @pl.when(k >= 2)
                    def _():
                        pltpu.make_async_copy(
                            rows.at[bg], acc.at[sidx.at[bg]], ssem[bg]).wait()

                    pltpu.async_copy(hp_hbm.at[srcb.at[bg]], rows.at[bg],
                                     gsem[bg])

                @pl.when(k + 2 < nch)
                def _():
                    fetch_idx(k + 2, bf, isem[bf])
            return 0

        lax.fori_loop(0, nch // 3, tri_body, 0)
        for sub in range(3):
            b = (nch - 3 + sub) % 3
            pltpu.make_async_copy(rows.at[b], acc.at[sidx.at[b]],
                                  ssem[b]).wait()
        plsc.subcore_barrier()
        pltpu.sync_copy(acc.at[pl.ds(rbase, ROWS_PER_TILE)],
                        out_hbm.at[c, pl.ds(rbase, ROWS_PER_TILE)])

    return spmm_kernel


_BLK = 512
_GRID = NP // _BLK

_row_spec = pl.BlockSpec((_BLK, D), lambda i: (i, 0))
_col_spec = pl.BlockSpec((_BLK, 1), lambda i: (i, 0))
_pair_spec = pl.BlockSpec((2, _BLK, D), lambda i: (0, i, 0))
_full_spec = lambda shape: pl.BlockSpec(shape, lambda i: (0,) * len(shape))


def _ln(x, w, b, eps=1e-5):
    mu = jnp.mean(x, axis=-1, keepdims=True)
    var = jnp.mean((x - mu) ** 2, axis=-1, keepdims=True)
    return (x - mu) * lax.rsqrt(var + eps) * w + b


def _tc_pre_body(x_ref, w1_ref, dp_ref, t1_ref, hp1_ref, dinv_ref):
    deg = dp_ref[0, :, 0:1] + dp_ref[1, :, 0:1] + 1.0
    dv = lax.rsqrt(deg)
    t = jnp.dot(x_ref[...], w1_ref[...], preferred_element_type=jnp.float32)
    t1_ref[...] = t
    hp1_ref[...] = t * dv
    dinv_ref[...] = dv


_tc_pre = pl.pallas_call(
    _tc_pre_body,
    grid=(_GRID,),
    in_specs=[_row_spec, _full_spec((D, D)), _pair_spec],
    out_specs=[_row_spec, _row_spec, _col_spec],
    out_shape=[
        jax.ShapeDtypeStruct((NP, D), jnp.float32),
        jax.ShapeDtypeStruct((NP, D), jnp.float32),
        jax.ShapeDtypeStruct((NP, 1), jnp.float32),
    ],
)


def _tc_mid_body(ap_ref, t1_ref, dinv_ref, b1_ref, lw_ref, lb_ref,
                 w2_ref, t2_ref, hp2_ref):
    dv = dinv_ref[...]
    pre = dv * (ap_ref[0] + ap_ref[1]) + dv * dv * t1_ref[...] + b1_ref[...]
    h1 = jnp.maximum(_ln(pre, lw_ref[...], lb_ref[...]), 0.0)
    t2 = jnp.dot(h1, w2_ref[...], preferred_element_type=jnp.float32)
    t2_ref[...] = t2
    hp2_ref[...] = t2 * dv


_tc_mid = pl.pallas_call(
    _tc_mid_body,
    grid=(_GRID,),
    in_specs=[_pair_spec, _row_spec, _col_spec,
              _full_spec((1, D)), _full_spec((1, D)), _full_spec((1, D)),
              _full_spec((D, D))],
    out_specs=[_row_spec, _row_spec],
    out_shape=[
        jax.ShapeDtypeStruct((NP, D), jnp.float32),
        jax.ShapeDtypeStruct((NP, D), jnp.float32),
    ],
)


def _tc_post_body(ap_ref, t2_ref, dinv_ref, b2_ref, l2w_ref, l2b_ref,
                  rw1_ref, rb1_ref, l3w_ref, l3b_ref, rw2_ref, rb2_ref, out_ref):
    dv = dinv_ref[...]
    pre = dv * (ap_ref[0] + ap_ref[1]) + dv * dv * t2_ref[...] + b2_ref[...]
    h2 = jnp.maximum(_ln(pre, l2w_ref[...], l2b_ref[...]), 0.0)
    r = jnp.dot(h2, rw1_ref[...], preferred_element_type=jnp.float32) + rb1_ref[...]
    r = jnp.maximum(_ln(r, l3w_ref[...], l3b_ref[...]), 0.0)
    out_ref[...] = jnp.dot(r, rw2_ref[...], preferred_element_type=jnp.float32) + rb2_ref[...]


_OBLK = 400
_out_row_spec = pl.BlockSpec((_OBLK, D), lambda i: (i, 0))
_out_col_spec = pl.BlockSpec((_OBLK, 1), lambda i: (i, 0))
_out_pair_spec = pl.BlockSpec((2, _OBLK, D), lambda i: (0, i, 0))

_tc_post = pl.pallas_call(
    _tc_post_body,
    grid=(N // _OBLK,),
    in_specs=[_out_pair_spec, _out_row_spec, _out_col_spec,
              _full_spec((1, D)), _full_spec((1, D)), _full_spec((1, D)),
              _full_spec((D, 32)), _full_spec((1, 32)), _full_spec((1, 32)),
              _full_spec((1, 32)), _full_spec((32, D)), _full_spec((1, D))],
    out_specs=_out_row_spec,
    out_shape=jax.ShapeDtypeStruct((N, D), jnp.float32),
)


def kernel(x, edge_index, edge_weight, W1, b1, ln1_w, ln1_b, W2, b2, ln2_w,
           ln2_b, rW1, rb1, ln3_w, ln3_b, rW2, rb2):
    e = edge_index.shape[1]
    step = NTILES * G * 6
    ep = -(-e // step) * step
    pad = ep - e

    src = edge_index[0].astype(jnp.int32)
    dst = edge_index[1].astype(jnp.int32)
    ew = edge_weight.astype(jnp.float32)
    if pad:
        padidx = (jnp.arange(pad, dtype=jnp.int32) * 37) % N
        src = jnp.concatenate([src, padidx])
        dst = jnp.concatenate([dst, padidx])
        ew = jnp.concatenate([ew, jnp.zeros((pad,), jnp.float32)])
    src2d = src.reshape(ep // G, G)
    dst2d = dst.reshape(ep // G, G)

    xp = jnp.pad(x.astype(jnp.float32), ((0, NP - N), (0, 0)))
    zrows = jnp.zeros((NP, D), jnp.float32)

    deg_k = _make_deg_kernel(ep)
    spmm_k = _make_spmm_kernel(ep)

    degp = deg_k(dst2d, ew, zrows)
    t1, hp1, dinv = _tc_pre(xp, W1, degp)

    agg1 = spmm_k(hp1, src2d, dst2d, ew, zrows)
    t2, hp2 = _tc_mid(agg1, t1, dinv, b1.reshape(1, D),
                      ln1_w.reshape(1, D), ln1_b.reshape(1, D), W2)

    agg2 = spmm_k(hp2, src2d, dst2d, ew, zrows)
    out = _tc_post(agg2, t2, dinv, b2.reshape(1, D),
                   ln2_w.reshape(1, D), ln2_b.reshape(1, D),
                   rW1, rb1.reshape(1, 32), ln3_w.reshape(1, 32),
                   ln3_b.reshape(1, 32), rW2, rb2.reshape(1, D))
    return out
